# Initial kernel scaffold; baseline (speedup 1.0000x reference)
import jax
import jax.numpy as jnp
from jax.experimental import pallas as pl


def kernel(x, pos_table):
    raise NotImplementedError("write your pallas kernel here")


# speedup vs baseline: 1.4558x; 1.4558x over previous
import jax
import jax.numpy as jnp
from jax.experimental import pallas as pl


_BLOCK_S = 512


def _body(x_ref, p_ref, o_ref):
    o_ref[...] = x_ref[...] + p_ref[...][None]


def kernel(x, pos_table):
    B, S, D = x.shape
    bs = min(_BLOCK_S, S)
    grid = (S // bs, B)
    return pl.pallas_call(
        _body,
        grid=grid,
        in_specs=[
            pl.BlockSpec((1, bs, D), lambda s, b: (b, s, 0)),
            pl.BlockSpec((bs, D), lambda s, b: (s, 0)),
        ],
        out_specs=pl.BlockSpec((1, bs, D), lambda s, b: (b, s, 0)),
        out_shape=jax.ShapeDtypeStruct(x.shape, x.dtype),
    )(x, pos_table)


# speedup vs baseline: 1.6746x; 1.1503x over previous
import jax
import jax.numpy as jnp
from jax.experimental import pallas as pl


_BLOCK_S = 1024


def _body(x_ref, p_ref, o_ref):
    o_ref[...] = x_ref[...] + p_ref[...][None]


def kernel(x, pos_table):
    B, S, D = x.shape
    bs = min(_BLOCK_S, S)
    grid = (S // bs, B)
    return pl.pallas_call(
        _body,
        grid=grid,
        in_specs=[
            pl.BlockSpec((1, bs, D), lambda s, b: (b, s, 0)),
            pl.BlockSpec((bs, D), lambda s, b: (s, 0)),
        ],
        out_specs=pl.BlockSpec((1, bs, D), lambda s, b: (b, s, 0)),
        out_shape=jax.ShapeDtypeStruct(x.shape, x.dtype),
    )(x, pos_table)


# speedup vs baseline: 1.7935x; 1.0710x over previous
import jax
import jax.numpy as jnp
from jax.experimental import pallas as pl


_BLOCK_S = 2048


def _body(x_ref, p_ref, o_ref):
    o_ref[...] = x_ref[...] + p_ref[...][None]


def kernel(x, pos_table):
    B, S, D = x.shape
    bs = min(_BLOCK_S, S)
    grid = (S // bs, B)
    return pl.pallas_call(
        _body,
        grid=grid,
        in_specs=[
            pl.BlockSpec((1, bs, D), lambda s, b: (b, s, 0)),
            pl.BlockSpec((bs, D), lambda s, b: (s, 0)),
        ],
        out_specs=pl.BlockSpec((1, bs, D), lambda s, b: (b, s, 0)),
        out_shape=jax.ShapeDtypeStruct(x.shape, x.dtype),
    )(x, pos_table)


# speedup vs baseline: 1.7990x; 1.0030x over previous
import jax
import jax.numpy as jnp
from jax.experimental import pallas as pl


_BLOCK_S = 1024


def _body(x_ref, p_ref, o_ref):
    o_ref[...] = x_ref[...] + p_ref[...][None]


def kernel(x, pos_table):
    B, S, D = x.shape
    bs = min(_BLOCK_S, S)
    grid = (S // bs,)
    return pl.pallas_call(
        _body,
        grid=grid,
        in_specs=[
            pl.BlockSpec((B, bs, D), lambda s: (0, s, 0)),
            pl.BlockSpec((bs, D), lambda s: (s, 0)),
        ],
        out_specs=pl.BlockSpec((B, bs, D), lambda s: (0, s, 0)),
        out_shape=jax.ShapeDtypeStruct(x.shape, x.dtype),
    )(x, pos_table)
